# Initial kernel scaffold; baseline (speedup 1.0000x reference)
import jax
import jax.numpy as jnp
from jax.experimental import pallas as pl


def kernel(samples, enc_weight, cent_weight):
    raise NotImplementedError("write your pallas kernel here")


# speedup vs baseline: 3.6302x; 3.6302x over previous
import functools

import jax
import jax.numpy as jnp
from jax.experimental import pallas as pl

B = 4096
F_IN = 512
D = 10000
C = 100

D_BLK = 2000
B_BLK = 512


def _stats_kernel(enc_ref, cent_t_ref, g_ref, k_ref, csq_ref):
    j = pl.program_id(0)
    e = enc_ref[...]
    ct = cent_t_ref[...]
    g = jax.lax.dot_general(e, e, (((0,), (0,)), ((), ())),
                            preferred_element_type=jnp.float32)
    k = jax.lax.dot_general(ct, e, (((0,), (0,)), ((), ())),
                            preferred_element_type=jnp.float32)
    csq = jnp.sum(ct * ct, axis=0, keepdims=True)

    @pl.when(j == 0)
    def _():
        g_ref[...] = g
        k_ref[...] = k
        csq_ref[...] = csq

    @pl.when(j > 0)
    def _():
        g_ref[...] += g
        k_ref[...] += k
        csq_ref[...] += csq


def _score_kernel(s_ref, g_ref, k_ref, csq_ref, out_ref):
    s = s_ref[...]
    t = jnp.dot(s, g_ref[...], preferred_element_type=jnp.float32)
    ssq = jnp.sum(t * s, axis=1, keepdims=True)
    raw = jax.lax.dot_general(s, k_ref[...], (((1,), (1,)), ((), ())),
                              preferred_element_type=jnp.float32)
    en = jnp.maximum(jnp.sqrt(ssq), 1e-12)
    cn = jnp.maximum(jnp.sqrt(csq_ref[...]), 1e-12)
    out_ref[...] = raw / en / cn


@functools.partial(jax.jit, static_argnames=("interpret",))
def kernel(samples, enc_weight, cent_weight, interpret=False):
    cent_t = cent_weight.T

    nd = D // D_BLK
    g, k, csq = pl.pallas_call(
        _stats_kernel,
        grid=(nd,),
        in_specs=[
            pl.BlockSpec((D_BLK, F_IN), lambda j: (j, 0)),
            pl.BlockSpec((D_BLK, C), lambda j: (j, 0)),
        ],
        out_specs=[
            pl.BlockSpec((F_IN, F_IN), lambda j: (0, 0)),
            pl.BlockSpec((C, F_IN), lambda j: (0, 0)),
            pl.BlockSpec((1, C), lambda j: (0, 0)),
        ],
        out_shape=[
            jax.ShapeDtypeStruct((F_IN, F_IN), jnp.float32),
            jax.ShapeDtypeStruct((C, F_IN), jnp.float32),
            jax.ShapeDtypeStruct((1, C), jnp.float32),
        ],
        interpret=interpret,
    )(enc_weight, cent_t)

    nb = B // B_BLK
    scores = pl.pallas_call(
        _score_kernel,
        grid=(nb,),
        in_specs=[
            pl.BlockSpec((B_BLK, F_IN), lambda i: (i, 0)),
            pl.BlockSpec((F_IN, F_IN), lambda i: (0, 0)),
            pl.BlockSpec((C, F_IN), lambda i: (0, 0)),
            pl.BlockSpec((1, C), lambda i: (0, 0)),
        ],
        out_specs=pl.BlockSpec((B_BLK, C), lambda i: (i, 0)),
        out_shape=jax.ShapeDtypeStruct((B, C), jnp.float32),
        interpret=interpret,
    )(samples, g, k, csq)
    return scores


# speedup vs baseline: 3.8549x; 1.0619x over previous
import functools

import jax
import jax.numpy as jnp
from jax.experimental import pallas as pl
from jax.experimental.pallas import tpu as pltpu

B = 4096
F_IN = 512
D = 10000
C = 100

D_BLK = 2000
B_BLK = 512
ND = D // D_BLK
NB = B // B_BLK


def _fused_kernel(enc_ref, ct_ref, s_ref, out_ref, g_ref, k_ref, csq_ref):
    t = pl.program_id(0)

    @pl.when(t < ND)
    def _stats():
        e = enc_ref[...]
        ct = ct_ref[...]
        g = jax.lax.dot_general(e, e, (((0,), (0,)), ((), ())),
                                preferred_element_type=jnp.float32)
        k = jax.lax.dot_general(ct, e, (((0,), (0,)), ((), ())),
                                preferred_element_type=jnp.float32)
        csq = jnp.sum(ct * ct, axis=0, keepdims=True)

        @pl.when(t == 0)
        def _():
            g_ref[...] = g
            k_ref[...] = k
            csq_ref[...] = csq

        @pl.when(t > 0)
        def _():
            g_ref[...] += g
            k_ref[...] += k
            csq_ref[...] += csq

    @pl.when(t >= ND)
    def _scores():
        s = s_ref[...]
        tt = jnp.dot(s, g_ref[...], preferred_element_type=jnp.float32)
        ssq = jnp.sum(tt * s, axis=1, keepdims=True)
        raw = jax.lax.dot_general(s, k_ref[...], (((1,), (1,)), ((), ())),
                                  preferred_element_type=jnp.float32)
        en = jnp.maximum(jnp.sqrt(ssq), 1e-12)
        cn = jnp.maximum(jnp.sqrt(csq_ref[...]), 1e-12)
        out_ref[...] = raw / en / cn


@functools.partial(jax.jit, static_argnames=("interpret",))
def kernel(samples, enc_weight, cent_weight, interpret=False):
    cent_t = cent_weight.T

    scores = pl.pallas_call(
        _fused_kernel,
        grid=(ND + NB,),
        in_specs=[
            pl.BlockSpec((D_BLK, F_IN), lambda t: (jnp.minimum(t, ND - 1), 0)),
            pl.BlockSpec((D_BLK, C), lambda t: (jnp.minimum(t, ND - 1), 0)),
            pl.BlockSpec((B_BLK, F_IN), lambda t: (jnp.maximum(t - ND, 0), 0)),
        ],
        out_specs=pl.BlockSpec((B_BLK, C), lambda t: (jnp.maximum(t - ND, 0), 0)),
        out_shape=jax.ShapeDtypeStruct((B, C), jnp.float32),
        scratch_shapes=[
            pltpu.VMEM((F_IN, F_IN), jnp.float32),
            pltpu.VMEM((C, F_IN), jnp.float32),
            pltpu.VMEM((1, C), jnp.float32),
        ],
        interpret=interpret,
    )(enc_weight, cent_t, samples)
    return scores


# speedup vs baseline: 5.3205x; 1.3802x over previous
import functools

import jax
import jax.numpy as jnp
from jax.experimental import pallas as pl
from jax.experimental.pallas import tpu as pltpu

B = 4096
F_IN = 512
D = 10000
C = 100

D_BLK = 2048
B_BLK = 1024
ND = (D + D_BLK - 1) // D_BLK
TAIL = D - (ND - 1) * D_BLK
NB = B // B_BLK


def _fused_kernel(enc_ref, cent_ref, s_ref, out_ref, g_ref, k_ref, csq_ref):
    t = pl.program_id(0)

    def stats(e, c):
        g = jax.lax.dot_general(e, e, (((0,), (0,)), ((), ())),
                                preferred_element_type=jnp.float32)
        k = jax.lax.dot_general(c, e, (((1,), (0,)), ((), ())),
                                preferred_element_type=jnp.float32)
        csq = jnp.sum(c * c, axis=1, keepdims=True)
        return g, k, csq

    @pl.when(t == 0)
    def _init():
        g, k, csq = stats(enc_ref[...], cent_ref[...])
        g_ref[...] = g
        k_ref[...] = k
        csq_ref[...] = csq

    @pl.when(jnp.logical_and(t > 0, t < ND - 1))
    def _accum():
        g, k, csq = stats(enc_ref[...], cent_ref[...])
        g_ref[...] += g
        k_ref[...] += k
        csq_ref[...] += csq

    @pl.when(t == ND - 1)
    def _accum_tail():
        e = enc_ref[...]
        c = cent_ref[...]
        rows = jax.lax.broadcasted_iota(jnp.int32, (D_BLK, 1), 0)
        e = jnp.where(rows < TAIL, e, 0.0)
        lanes = jax.lax.broadcasted_iota(jnp.int32, (1, D_BLK), 1)
        c = jnp.where(lanes < TAIL, c, 0.0)
        g, k, csq = stats(e, c)
        g_ref[...] += g
        csq = csq_ref[...] + csq
        cn = jnp.maximum(jnp.sqrt(csq), 1e-12)
        k_ref[...] = (k_ref[...] + k) / cn

    @pl.when(t >= ND)
    def _scores():
        s = s_ref[...]
        tt = jnp.dot(s, g_ref[...], preferred_element_type=jnp.float32)
        ssq = jnp.sum(tt * s, axis=1, keepdims=True)
        raw = jax.lax.dot_general(s, k_ref[...], (((1,), (1,)), ((), ())),
                                  preferred_element_type=jnp.float32)
        en = jnp.maximum(jnp.sqrt(ssq), 1e-12)
        out_ref[...] = raw / en


@functools.partial(jax.jit, static_argnames=("interpret",))
def kernel(samples, enc_weight, cent_weight, interpret=False):
    scores = pl.pallas_call(
        _fused_kernel,
        grid=(ND + NB,),
        in_specs=[
            pl.BlockSpec((D_BLK, F_IN), lambda t: (jnp.minimum(t, ND - 1), 0)),
            pl.BlockSpec((C, D_BLK), lambda t: (0, jnp.minimum(t, ND - 1))),
            pl.BlockSpec((B_BLK, F_IN), lambda t: (jnp.maximum(t - ND, 0), 0)),
        ],
        out_specs=pl.BlockSpec((B_BLK, C), lambda t: (jnp.maximum(t - ND, 0), 0)),
        out_shape=jax.ShapeDtypeStruct((B, C), jnp.float32),
        scratch_shapes=[
            pltpu.VMEM((F_IN, F_IN), jnp.float32),
            pltpu.VMEM((C, F_IN), jnp.float32),
            pltpu.VMEM((C, 1), jnp.float32),
        ],
        interpret=interpret,
    )(enc_weight, cent_weight, samples)
    return scores


# speedup vs baseline: 5.3239x; 1.0006x over previous
import functools

import jax
import jax.numpy as jnp
from jax.experimental import pallas as pl
from jax.experimental.pallas import tpu as pltpu

B = 4096
F_IN = 512
D = 10000
C = 100

D_BLK = 2048
B_BLK = 1024
ND = (D + D_BLK - 1) // D_BLK
TAIL = D - (ND - 1) * D_BLK
NB = B // B_BLK


def _fused_kernel(enc_ref, cent_ref, s_ref, out_ref, g_ref, k_ref, csq_ref):
    t = pl.program_id(0)

    def stats(e, c):
        eb = e.astype(jnp.bfloat16)
        cb = c.astype(jnp.bfloat16)
        g = jax.lax.dot_general(eb, eb, (((0,), (0,)), ((), ())),
                                preferred_element_type=jnp.float32)
        k = jax.lax.dot_general(cb, eb, (((1,), (0,)), ((), ())),
                                preferred_element_type=jnp.float32)
        csq = jnp.sum(c * c, axis=1, keepdims=True)
        return g, k, csq

    @pl.when(t == 0)
    def _init():
        g, k, csq = stats(enc_ref[...], cent_ref[...])
        g_ref[...] = g
        k_ref[...] = k
        csq_ref[...] = csq

    @pl.when(jnp.logical_and(t > 0, t < ND - 1))
    def _accum():
        g, k, csq = stats(enc_ref[...], cent_ref[...])
        g_ref[...] += g
        k_ref[...] += k
        csq_ref[...] += csq

    @pl.when(t == ND - 1)
    def _accum_tail():
        e = enc_ref[...]
        c = cent_ref[...]
        rows = jax.lax.broadcasted_iota(jnp.int32, (D_BLK, 1), 0)
        e = jnp.where(rows < TAIL, e, 0.0)
        lanes = jax.lax.broadcasted_iota(jnp.int32, (1, D_BLK), 1)
        c = jnp.where(lanes < TAIL, c, 0.0)
        g, k, csq = stats(e, c)
        g_ref[...] += g
        csq = csq_ref[...] + csq
        cn = jnp.maximum(jnp.sqrt(csq), 1e-12)
        k_ref[...] = (k_ref[...] + k) / cn

    @pl.when(t >= ND)
    def _scores():
        s = s_ref[...]
        sb = s.astype(jnp.bfloat16)
        tt = jnp.dot(sb, g_ref[...].astype(jnp.bfloat16),
                     preferred_element_type=jnp.float32)
        ssq = jnp.sum(tt * s, axis=1, keepdims=True)
        raw = jax.lax.dot_general(sb, k_ref[...].astype(jnp.bfloat16),
                                  (((1,), (1,)), ((), ())),
                                  preferred_element_type=jnp.float32)
        en = jnp.maximum(jnp.sqrt(ssq), 1e-12)
        out_ref[...] = raw / en


@functools.partial(jax.jit, static_argnames=("interpret",))
def kernel(samples, enc_weight, cent_weight, interpret=False):
    scores = pl.pallas_call(
        _fused_kernel,
        grid=(ND + NB,),
        in_specs=[
            pl.BlockSpec((D_BLK, F_IN), lambda t: (jnp.minimum(t, ND - 1), 0)),
            pl.BlockSpec((C, D_BLK), lambda t: (0, jnp.minimum(t, ND - 1))),
            pl.BlockSpec((B_BLK, F_IN), lambda t: (jnp.maximum(t - ND, 0), 0)),
        ],
        out_specs=pl.BlockSpec((B_BLK, C), lambda t: (jnp.maximum(t - ND, 0), 0)),
        out_shape=jax.ShapeDtypeStruct((B, C), jnp.float32),
        scratch_shapes=[
            pltpu.VMEM((F_IN, F_IN), jnp.float32),
            pltpu.VMEM((C, F_IN), jnp.float32),
            pltpu.VMEM((C, 1), jnp.float32),
        ],
        interpret=interpret,
    )(enc_weight, cent_weight, samples)
    return scores
